# baseline (speedup 1.0000x reference)
import functools

import jax
import jax.numpy as jnp
from jax import lax
from jax.experimental import pallas as pl
from jax.experimental.pallas import tpu as pltpu
from jax.experimental.pallas import tpu_sc as plsc

N = 10000
E = 320000
D = 128
H = 256
G = 64
C = 10

_K = 64
_NSUB = 16
_NCORE = 2
_EPS = E // 32
_NCH = 158
_EPAD = _NCH * _K - _EPS
_ACC_ROWS = N + _EPAD
_RPS = 624
_REM = N - _RPS * _NSUB


def _make_sc_agg(nch: int):
  mesh = plsc.VectorSubcoreMesh(core_axis_name="c", subcore_axis_name="s")

  @functools.partial(
      pl.kernel,
      mesh=mesh,
      out_type=jax.ShapeDtypeStruct((_NCORE, N, D), jnp.float32),
      scratch_types=[
          pltpu.VMEM(((nch + 1) * _K,), jnp.int32),
          pltpu.VMEM((nch, _K), jnp.int32),
          pltpu.VMEM((_K, D), jnp.float32),
          pltpu.VMEM((_K, D), jnp.float32),
          pltpu.VMEM_SHARED((_ACC_ROWS, D), jnp.float32),
          pltpu.SemaphoreType.DMA,
          pltpu.SemaphoreType.DMA,
      ],
  )
  def sc_agg(h_hbm, srcT_hbm, dstT_hbm, zz_hbm, out_hbm,
             srcv, dstv, buf0, buf1, acc, sem0, sem1):
    c = lax.axis_index("c")
    s = lax.axis_index("s")
    pltpu.sync_copy(srcT_hbm.at[c, s], srcv)
    pltpu.sync_copy(dstT_hbm.at[c, s], dstv)
    pltpu.async_copy(h_hbm.at[srcv.at[pl.ds(0, _K)]], buf0, sem0)
    pltpu.sync_copy(zz_hbm.at[pl.ds(s * _RPS, _RPS)],
                    acc.at[pl.ds(s * _RPS, _RPS)])

    @pl.when(s == 0)
    def _():
      pltpu.sync_copy(zz_hbm.at[pl.ds(_RPS * _NSUB, _REM)],
                      acc.at[pl.ds(_RPS * _NSUB, _REM)])

    plsc.subcore_barrier()

    def body(i, carry):
      j0 = 2 * i
      j1 = j0 + 1
      pltpu.make_async_copy(h_hbm.at[srcv.at[pl.ds(j0 * _K, _K)]],
                            buf0, sem0).wait()
      pltpu.async_copy(h_hbm.at[srcv.at[pl.ds(j1 * _K, _K)]], buf1, sem1)
      pltpu.sync_copy(buf0, acc.at[dstv.at[j0]], add=True)
      pltpu.make_async_copy(h_hbm.at[srcv.at[pl.ds(j1 * _K, _K)]],
                            buf1, sem1).wait()
      pltpu.async_copy(h_hbm.at[srcv.at[pl.ds((j0 + 2) * _K, _K)]],
                       buf0, sem0)
      pltpu.sync_copy(buf1, acc.at[dstv.at[j1]], add=True)
      return carry

    lax.fori_loop(0, nch // 2, body, 0)
    pltpu.make_async_copy(h_hbm.at[srcv.at[pl.ds(0, _K)]], buf0, sem0).wait()
    plsc.subcore_barrier()
    pltpu.sync_copy(acc.at[pl.ds(s * _RPS, _RPS)],
                    out_hbm.at[c, pl.ds(s * _RPS, _RPS)])

    @pl.when(s == 0)
    def _():
      pltpu.sync_copy(acc.at[pl.ds(_RPS * _NSUB, _REM)],
                      out_hbm.at[c, pl.ds(_RPS * _NSUB, _REM)])

  return sc_agg


_sc_agg_l0 = _make_sc_agg(_NCH)


def _mlp0_body(h_ref, agg_ref, w1_ref, b1_ref, w2_ref, b2_ref, eps_ref, o_ref):
  z = (1.0 + eps_ref[0, 0]) * h_ref[...] + agg_ref[0] + agg_ref[1]
  y = jnp.maximum(jnp.dot(z, w1_ref[...],
                          preferred_element_type=jnp.float32) + b1_ref[...], 0.0)
  o_ref[...] = jnp.maximum(jnp.dot(y, w2_ref[...],
                                   preferred_element_type=jnp.float32)
                           + b2_ref[...], 0.0)


def _mlp1_body(h_ref, aggl_ref, aggr_ref, w1_ref, b1_ref, w2_ref, b2_ref,
               eps_ref, o_ref):
  agg = jnp.concatenate([aggl_ref[0] + aggl_ref[1],
                         aggr_ref[0] + aggr_ref[1]], axis=-1)
  z = (1.0 + eps_ref[0, 0]) * h_ref[...] + agg
  y = jnp.maximum(jnp.dot(z, w1_ref[...],
                          preferred_element_type=jnp.float32) + b1_ref[...], 0.0)
  o_ref[...] = jnp.maximum(jnp.dot(y, w2_ref[...],
                                   preferred_element_type=jnp.float32)
                           + b2_ref[...], 0.0)


_BR = 2000


def _tc_mlp(h, aggs, w1, b1, w2, b2, eps, body, din):
  nblk = N // _BR
  agg_specs = [pl.BlockSpec((2, _BR, D), lambda i: (0, i, 0)) for _ in aggs]
  return pl.pallas_call(
      body,
      grid=(nblk,),
      in_specs=[
          pl.BlockSpec((_BR, din), lambda i: (i, 0)),
          *agg_specs,
          pl.BlockSpec((din, H), lambda i: (0, 0)),
          pl.BlockSpec((1, H), lambda i: (0, 0)),
          pl.BlockSpec((H, H), lambda i: (0, 0)),
          pl.BlockSpec((1, H), lambda i: (0, 0)),
          pl.BlockSpec((1, 1), lambda i: (0, 0)),
      ],
      out_specs=pl.BlockSpec((_BR, H), lambda i: (i, 0)),
      out_shape=jax.ShapeDtypeStruct((N, H), jnp.float32),
  )(h, *aggs, w1, b1.reshape(1, H), w2, b2.reshape(1, H), eps.reshape(1, 1))


def _pool_body(h_ref, batch_ref, wf_ref, bf_ref, o_ref, acc_ref):
  i = pl.program_id(0)

  @pl.when(i == 0)
  def _():
    acc_ref[...] = jnp.zeros_like(acc_ref)

  b = batch_ref[0, 0, :]
  gids = lax.broadcasted_iota(jnp.int32, (G, _BR), 0)
  mask = (b[None, :] == gids).astype(jnp.float32)
  acc_ref[...] += jnp.dot(mask, h_ref[...], preferred_element_type=jnp.float32)

  @pl.when(i == pl.num_programs(0) - 1)
  def _():
    logits = jnp.dot(acc_ref[...], wf_ref[...],
                     preferred_element_type=jnp.float32) + bf_ref[...]
    m = jnp.max(logits, axis=1, keepdims=True)
    shifted = logits - m
    lse = jnp.log(jnp.sum(jnp.exp(shifted), axis=1, keepdims=True))
    o_ref[...] = shifted - lse


def _tc_pool(h, batch, wf, bf):
  nblk = N // _BR
  return pl.pallas_call(
      _pool_body,
      grid=(nblk,),
      in_specs=[
          pl.BlockSpec((_BR, H), lambda i: (i, 0)),
          pl.BlockSpec((1, 1, _BR), lambda i: (i, 0, 0)),
          pl.BlockSpec((H, C), lambda i: (0, 0)),
          pl.BlockSpec((1, C), lambda i: (0, 0)),
      ],
      out_specs=pl.BlockSpec((G, C), lambda i: (0, 0)),
      out_shape=jax.ShapeDtypeStruct((G, C), jnp.float32),
      scratch_shapes=[pltpu.VMEM((G, H), jnp.float32)],
  )(h, batch.reshape(nblk, 1, _BR), wf, bf.reshape(1, C))


def kernel(x, edge_index, batch, W1, b1, W2, b2, eps0, W3, b3, W4, b4, eps1,
           Wf, bf):
  src = edge_index[0]
  dst = edge_index[1]
  zz = jnp.zeros((N, D), jnp.float32)

  src0 = jnp.pad(src.reshape(32, _EPS),
                 ((0, 0), (0, _EPAD + _K))).reshape(_NCORE, _NSUB,
                                                    (_NCH + 1) * _K)
  dump = jnp.broadcast_to(N + jnp.arange(_EPAD, dtype=jnp.int32), (32, _EPAD))
  dst0 = jnp.concatenate([dst.reshape(32, _EPS), dump],
                         axis=1).reshape(_NCORE, _NSUB, _NCH, _K)

  agg0 = _sc_agg_l0(x, src0, dst0, zz)
  h1 = _tc_mlp(x, [agg0], W1, b1, W2, b2, eps0, _mlp0_body, D)

  aggL = _sc_agg_l0(h1[:, :D], src0, dst0, zz)
  aggR = _sc_agg_l0(h1[:, D:], src0, dst0, zz)
  h2 = _tc_mlp(h1, [aggL, aggR], W3, b3, W4, b4, eps1, _mlp1_body, H)

  return _tc_pool(h2, batch, Wf, bf)


# speedup vs baseline: 1.7496x; 1.7496x over previous
import functools

import jax
import jax.numpy as jnp
from jax import lax
from jax.experimental import pallas as pl
from jax.experimental.pallas import tpu as pltpu
from jax.experimental.pallas import tpu_sc as plsc

N = 10000
E = 320000
D = 128
H = 256
G = 64
C = 10

_K = 80
_NSUB = 16
_NCORE = 2
_RPS = 624
_REM = N - _RPS * _NSUB


def _make_sc_agg(num_h_rows: int, nch: int):
  mesh = plsc.VectorSubcoreMesh(core_axis_name="c", subcore_axis_name="s")

  @functools.partial(
      pl.kernel,
      mesh=mesh,
      out_type=jax.ShapeDtypeStruct((_NCORE, N, D), jnp.float32),
      scratch_types=[
          pltpu.VMEM((nch, _K), jnp.int32),
          pltpu.VMEM((nch, _K), jnp.int32),
          pltpu.VMEM((_K, D), jnp.float32),
          pltpu.VMEM_SHARED((N, D), jnp.float32),
          pltpu.SemaphoreType.DMA,
      ],
  )
  def sc_agg(h_hbm, srcT_hbm, dstT_hbm, zz_hbm, out_hbm,
             srcv, dstv, rows, acc, sem):
    c = lax.axis_index("c")
    s = lax.axis_index("s")
    pltpu.sync_copy(zz_hbm.at[pl.ds(s * _RPS, _RPS)],
                    acc.at[pl.ds(s * _RPS, _RPS)])

    @pl.when(s == 0)
    def _():
      pltpu.sync_copy(zz_hbm.at[pl.ds(_RPS * _NSUB, _REM)],
                      acc.at[pl.ds(_RPS * _NSUB, _REM)])
    pltpu.sync_copy(srcT_hbm.at[c, s], srcv)
    pltpu.sync_copy(dstT_hbm.at[c, s], dstv)
    plsc.subcore_barrier()

    def body(j, carry):
      pltpu.async_copy(h_hbm.at[srcv.at[j]], rows, sem).wait()
      pltpu.sync_copy(rows, acc.at[dstv.at[j]], add=True)
      return carry

    lax.fori_loop(0, nch, body, 0)
    plsc.subcore_barrier()
    pltpu.sync_copy(acc.at[pl.ds(s * _RPS, _RPS)],
                    out_hbm.at[c, pl.ds(s * _RPS, _RPS)])

    @pl.when(s == 0)
    def _():
      pltpu.sync_copy(acc.at[pl.ds(_RPS * _NSUB, _REM)],
                      out_hbm.at[c, pl.ds(_RPS * _NSUB, _REM)])

  return sc_agg


_sc_agg_l0 = _make_sc_agg(N, (E // 32) // _K)


def _mlp0_body(h_ref, agg_ref, w1_ref, b1_ref, w2_ref, b2_ref, eps_ref, o_ref):
  z = (1.0 + eps_ref[0, 0]) * h_ref[...] + agg_ref[0] + agg_ref[1]
  y = jnp.maximum(jnp.dot(z, w1_ref[...],
                          preferred_element_type=jnp.float32) + b1_ref[...], 0.0)
  o_ref[...] = jnp.maximum(jnp.dot(y, w2_ref[...],
                                   preferred_element_type=jnp.float32)
                           + b2_ref[...], 0.0)


def _mlp1_pool_body(h_ref, aggl_ref, aggr_ref, w1_ref, b1_ref, w2_ref,
                    b2_ref, eps_ref, batch_ref, wf_ref, bf_ref, o_ref,
                    acc_ref):
  i = pl.program_id(0)

  @pl.when(i == 0)
  def _():
    acc_ref[...] = jnp.zeros_like(acc_ref)

  agg = jnp.concatenate([aggl_ref[0] + aggl_ref[1],
                         aggr_ref[0] + aggr_ref[1]], axis=-1)
  z = (1.0 + eps_ref[0, 0]) * h_ref[...] + agg
  y = jnp.maximum(jnp.dot(z, w1_ref[...],
                          preferred_element_type=jnp.float32) + b1_ref[...], 0.0)
  h2 = jnp.maximum(jnp.dot(y, w2_ref[...],
                           preferred_element_type=jnp.float32)
                   + b2_ref[...], 0.0)
  b = batch_ref[0, 0, :]
  gids = lax.broadcasted_iota(jnp.int32, (G, _BR), 0)
  mask = (b[None, :] == gids).astype(jnp.float32)
  acc_ref[...] += jnp.dot(mask, h2, preferred_element_type=jnp.float32)

  @pl.when(i == pl.num_programs(0) - 1)
  def _():
    logits = jnp.dot(acc_ref[...], wf_ref[...],
                     preferred_element_type=jnp.float32) + bf_ref[...]
    m = jnp.max(logits, axis=1, keepdims=True)
    shifted = logits - m
    lse = jnp.log(jnp.sum(jnp.exp(shifted), axis=1, keepdims=True))
    o_ref[...] = shifted - lse


_BR = 2000


def _tc_mlp(h, aggs, w1, b1, w2, b2, eps, body, din):
  nblk = N // _BR
  agg_specs = [pl.BlockSpec((2, _BR, D), lambda i: (0, i, 0)) for _ in aggs]
  return pl.pallas_call(
      body,
      grid=(nblk,),
      in_specs=[
          pl.BlockSpec((_BR, din), lambda i: (i, 0)),
          *agg_specs,
          pl.BlockSpec((din, H), lambda i: (0, 0)),
          pl.BlockSpec((1, H), lambda i: (0, 0)),
          pl.BlockSpec((H, H), lambda i: (0, 0)),
          pl.BlockSpec((1, H), lambda i: (0, 0)),
          pl.BlockSpec((1, 1), lambda i: (0, 0)),
      ],
      out_specs=pl.BlockSpec((_BR, H), lambda i: (i, 0)),
      out_shape=jax.ShapeDtypeStruct((N, H), jnp.float32),
  )(h, *aggs, w1, b1.reshape(1, H), w2, b2.reshape(1, H), eps.reshape(1, 1))


def _tc_mlp1_pool(h, aggs, w1, b1, w2, b2, eps, batch, wf, bf):
  nblk = N // _BR
  agg_specs = [pl.BlockSpec((2, _BR, D), lambda i: (0, i, 0)) for _ in aggs]
  return pl.pallas_call(
      _mlp1_pool_body,
      grid=(nblk,),
      in_specs=[
          pl.BlockSpec((_BR, H), lambda i: (i, 0)),
          *agg_specs,
          pl.BlockSpec((H, H), lambda i: (0, 0)),
          pl.BlockSpec((1, H), lambda i: (0, 0)),
          pl.BlockSpec((H, H), lambda i: (0, 0)),
          pl.BlockSpec((1, H), lambda i: (0, 0)),
          pl.BlockSpec((1, 1), lambda i: (0, 0)),
          pl.BlockSpec((1, 1, _BR), lambda i: (i, 0, 0)),
          pl.BlockSpec((H, C), lambda i: (0, 0)),
          pl.BlockSpec((1, C), lambda i: (0, 0)),
      ],
      out_specs=pl.BlockSpec((G, C), lambda i: (0, 0)),
      out_shape=jax.ShapeDtypeStruct((G, C), jnp.float32),
      scratch_shapes=[pltpu.VMEM((G, H), jnp.float32)],
  )(h, *aggs, w1, b1.reshape(1, H), w2, b2.reshape(1, H), eps.reshape(1, 1),
    batch.reshape(nblk, 1, _BR), wf, bf.reshape(1, C))


def kernel(x, edge_index, batch, W1, b1, W2, b2, eps0, W3, b3, W4, b4, eps1,
           Wf, bf):
  src = edge_index[0]
  dst = edge_index[1]
  zz = jnp.zeros((N, D), jnp.float32)

  src0 = src.reshape(_NCORE, _NSUB, -1, _K)
  dst0 = dst.reshape(_NCORE, _NSUB, -1, _K)
  agg0 = _sc_agg_l0(x, src0, dst0, zz)
  h1 = _tc_mlp(x, [agg0], W1, b1, W2, b2, eps0, _mlp0_body, D)

  aggL = _sc_agg_l0(h1[:, :D], src0, dst0, zz)
  aggR = _sc_agg_l0(h1[:, D:], src0, dst0, zz)

  return _tc_mlp1_pool(h1, [aggL, aggR], W3, b3, W4, b4, eps1, batch, Wf, bf)
